# Initial kernel scaffold; baseline (speedup 1.0000x reference)
import jax
import jax.numpy as jnp
from jax import lax
from jax.experimental import pallas as pl

_B = 128
_QS = 48
_CS = 64
_M = 64
_PAIR = _QS + _CS
_DEG = 16
_EP = _PAIR * _DEG
_N_PROP = 3
_N_OUTER = 5
_SINK_ITERS = 10
_TEMP = 0.1


def _pair_body(nf_ref, ef_ref, fic_ref, tic_ref, tir_ref,
               Wn_ref, bn_ref, We_ref, be_ref, Wm1_ref, bm1_ref, Wm2_ref,
               bm2_ref, Wu1_ref, bu1_ref, Wu2_ref, bu2_ref, W1_ref, b1_ref,
               W2_ref, b2_ref, out_ref):
    b = pl.program_id(0)
    base = b * _PAIR

    nfb = nf_ref[...]
    efb = ef_ref[...]
    fic = fic_ref[0] - base
    tic = tic_ref[0] - base
    tir = tir_ref[0] - base

    e_iota = lax.broadcasted_iota(jnp.int32, (_EP, _PAIR), 1)
    F = (fic == e_iota).astype(jnp.float32)
    T = (tic == e_iota).astype(jnp.float32)
    n_iota = lax.broadcasted_iota(jnp.int32, (_PAIR, _EP), 0)
    TTo = (tir == n_iota).astype(jnp.float32)

    Wm1 = Wm1_ref[...]
    Wm1a = Wm1[0:64, :]
    Wm1b = Wm1[64:128, :]
    Wm1e = Wm1[128:144, :]
    Wm2 = Wm2_ref[...]
    bm2 = bm2_ref[...]
    Wu1 = Wu1_ref[...]
    Wu1h = Wu1[0:64, :]
    Wu1a = Wu1[64:128, :]
    bu1 = bu1_ref[...]
    Wu2 = Wu2_ref[...]
    bu2 = bu2_ref[...]
    W1 = W1_ref[...]
    b1 = b1_ref[...]
    W2 = W2_ref[...]
    b2 = b2_ref[...]

    h0 = jnp.dot(nfb, Wn_ref[...], preferred_element_type=jnp.float32) + bn_ref[...]
    e_lin = jnp.dot(efb, We_ref[...], preferred_element_type=jnp.float32) + be_ref[...]
    e_term = jnp.dot(e_lin, Wm1e, preferred_element_type=jnp.float32) + bm1_ref[...]

    qmask = (lax.broadcasted_iota(jnp.int32, (_M, 1), 0) < _QS).astype(jnp.float32)
    mask_col = jnp.ones((_PAIR, 1), jnp.float32)

    qp = cc = plan = None
    for _j in range(_N_OUTER):
        h = h0
        for _i in range(_N_PROP):
            hA = jnp.dot(h, Wm1a, preferred_element_type=jnp.float32)
            hB = jnp.dot(h, Wm1b, preferred_element_type=jnp.float32)
            pre = (jnp.dot(F, hA, preferred_element_type=jnp.float32)
                   + jnp.dot(T, hB, preferred_element_type=jnp.float32)
                   + e_term)
            msg = jnp.dot(jnp.maximum(pre, 0.0), Wm2,
                          preferred_element_type=jnp.float32) + bm2
            mf = jnp.dot(F, mask_col, preferred_element_type=jnp.float32)
            msg = msg * mf
            agg = jnp.dot(TTo, msg, preferred_element_type=jnp.float32)
            u = (jnp.dot(h, Wu1h, preferred_element_type=jnp.float32)
                 + jnp.dot(agg, Wu1a, preferred_element_type=jnp.float32) + bu1)
            h = jnp.dot(jnp.maximum(u, 0.0), Wu2,
                        preferred_element_type=jnp.float32) + bu2

        qp = h[0:_M, :] * qmask
        cc = h[_QS:_PAIR, :]
        tq = jnp.dot(jnp.maximum(jnp.dot(qp, W1, preferred_element_type=jnp.float32) + b1, 0.0),
                     W2, preferred_element_type=jnp.float32) + b2
        tc = jnp.dot(jnp.maximum(jnp.dot(cc, W1, preferred_element_type=jnp.float32) + b1, 0.0),
                     W2, preferred_element_type=jnp.float32) + b2
        mq = tq * qmask
        sink = lax.dot_general(mq, tc, (((1,), (1,)), ((), ())),
                               preferred_element_type=jnp.float32)
        la = sink / _TEMP
        for _s in range(_SINK_ITERS):
            mx = jnp.max(la, axis=1, keepdims=True)
            la = la - (mx + jnp.log(jnp.sum(jnp.exp(la - mx), axis=1, keepdims=True)))
            mx = jnp.max(la, axis=0, keepdims=True)
            la = la - (mx + jnp.log(jnp.sum(jnp.exp(la - mx), axis=0, keepdims=True)))
        plan = jnp.exp(la)
        temp_col = jnp.sum(plan, axis=1, keepdims=True) * qmask
        mask_col = jnp.concatenate(
            [jnp.ones((_QS, 1), jnp.float32), temp_col], axis=0)

    pc = jnp.dot(plan, cc, preferred_element_type=jnp.float32)
    score = -jnp.sum(jnp.maximum(qp - pc, 0.0))
    out_ref[...] = jnp.full((1, 128), score, jnp.float32)


def kernel(node_features, edge_features, Wn, bn, We, be, Wm1, bm1, Wm2, bm2,
           Wu1, bu1, Wu2, bu2, W1, b1, W2, b2, from_idx, to_idx, graph_idx):
    del graph_idx
    fic = from_idx.reshape(_B, _EP, 1)
    tic = to_idx.reshape(_B, _EP, 1)
    tir = to_idx.reshape(_B, 1, _EP)

    def row(v):
        return v.reshape(1, -1)

    grid_spec = pl.GridSpec(
        grid=(_B,),
        in_specs=[
            pl.BlockSpec((_PAIR, 32), lambda b: (b, 0)),
            pl.BlockSpec((_EP, 16), lambda b: (b, 0)),
            pl.BlockSpec((1, _EP, 1), lambda b: (b, 0, 0)),
            pl.BlockSpec((1, _EP, 1), lambda b: (b, 0, 0)),
            pl.BlockSpec((1, 1, _EP), lambda b: (b, 0, 0)),
            pl.BlockSpec((32, 64), lambda b: (0, 0)),
            pl.BlockSpec((1, 64), lambda b: (0, 0)),
            pl.BlockSpec((16, 16), lambda b: (0, 0)),
            pl.BlockSpec((1, 16), lambda b: (0, 0)),
            pl.BlockSpec((144, 64), lambda b: (0, 0)),
            pl.BlockSpec((1, 64), lambda b: (0, 0)),
            pl.BlockSpec((64, 64), lambda b: (0, 0)),
            pl.BlockSpec((1, 64), lambda b: (0, 0)),
            pl.BlockSpec((128, 64), lambda b: (0, 0)),
            pl.BlockSpec((1, 64), lambda b: (0, 0)),
            pl.BlockSpec((64, 64), lambda b: (0, 0)),
            pl.BlockSpec((1, 64), lambda b: (0, 0)),
            pl.BlockSpec((64, 64), lambda b: (0, 0)),
            pl.BlockSpec((1, 64), lambda b: (0, 0)),
            pl.BlockSpec((64, 64), lambda b: (0, 0)),
            pl.BlockSpec((1, 64), lambda b: (0, 0)),
        ],
        out_specs=pl.BlockSpec((1, 128), lambda b: (b, 0)),
    )
    out = pl.pallas_call(
        _pair_body,
        grid_spec=grid_spec,
        out_shape=jax.ShapeDtypeStruct((_B, 128), jnp.float32),
    )(node_features, edge_features, fic, tic, tir,
      Wn, row(bn), We, row(be), Wm1, row(bm1), Wm2, row(bm2),
      Wu1, row(bu1), Wu2, row(bu2), W1, row(b1), W2, row(b2))
    return out[:, 0]


# speedup vs baseline: 2.5720x; 2.5720x over previous
import jax
import jax.numpy as jnp
from jax import lax
from jax.experimental import pallas as pl

_B = 128
_QS = 48
_CS = 64
_M = 64
_PAIR = _QS + _CS
_DEG = 16
_EP = _PAIR * _DEG
_N_PROP = 3
_N_OUTER = 5
_SINK_ITERS = 10
_TEMP = 0.1
_PREC = lax.Precision.HIGHEST


def _pair_body(nf_ref, ef_ref, fic_ref, tic_ref, tir_ref,
               Wn_ref, bn_ref, We_ref, be_ref, Wm1_ref, bm1_ref, Wm2_ref,
               bm2_ref, Wu1_ref, bu1_ref, Wu2_ref, bu2_ref, W1_ref, b1_ref,
               W2_ref, b2_ref, out_ref):
    b = pl.program_id(0)
    base = b * _PAIR

    nfb = nf_ref[...]
    efb = ef_ref[...]
    fic = fic_ref[0] - base
    tic = tic_ref[0] - base
    tir = tir_ref[0] - base

    e_iota = lax.broadcasted_iota(jnp.int32, (_EP, _PAIR), 1)
    F = (fic == e_iota).astype(jnp.float32)
    T = (tic == e_iota).astype(jnp.float32)
    n_iota = lax.broadcasted_iota(jnp.int32, (_PAIR, _EP), 0)
    TTo = (tir == n_iota).astype(jnp.float32)

    Wm1 = Wm1_ref[...]
    Wm1a = Wm1[0:64, :]
    Wm1b = Wm1[64:128, :]
    Wm1e = Wm1[128:144, :]
    Wm2 = Wm2_ref[...]
    bm2 = bm2_ref[...]
    Wu1 = Wu1_ref[...]
    Wu1h = Wu1[0:64, :]
    Wu1a = Wu1[64:128, :]
    bu1 = bu1_ref[...]
    Wu2 = Wu2_ref[...]
    bu2 = bu2_ref[...]
    W1 = W1_ref[...]
    b1 = b1_ref[...]
    W2 = W2_ref[...]
    b2 = b2_ref[...]

    h0 = jnp.dot(nfb, Wn_ref[...], preferred_element_type=jnp.float32, precision=_PREC) + bn_ref[...]
    e_lin = jnp.dot(efb, We_ref[...], preferred_element_type=jnp.float32, precision=_PREC) + be_ref[...]
    e_term = jnp.dot(e_lin, Wm1e, preferred_element_type=jnp.float32, precision=_PREC) + bm1_ref[...]

    qmask = (lax.broadcasted_iota(jnp.int32, (_M, 1), 0) < _QS).astype(jnp.float32)
    mask_col = jnp.ones((_PAIR, 1), jnp.float32)

    qp = cc = plan = None
    for _j in range(_N_OUTER):
        h = h0
        for _i in range(_N_PROP):
            hA = jnp.dot(h, Wm1a, preferred_element_type=jnp.float32, precision=_PREC)
            hB = jnp.dot(h, Wm1b, preferred_element_type=jnp.float32, precision=_PREC)
            pre = (jnp.dot(F, hA, preferred_element_type=jnp.float32, precision=_PREC)
                   + jnp.dot(T, hB, preferred_element_type=jnp.float32, precision=_PREC)
                   + e_term)
            msg = jnp.dot(jnp.maximum(pre, 0.0), Wm2,
                          preferred_element_type=jnp.float32, precision=_PREC) + bm2
            mf = jnp.dot(F, mask_col, preferred_element_type=jnp.float32, precision=_PREC)
            msg = msg * mf
            agg = jnp.dot(TTo, msg, preferred_element_type=jnp.float32, precision=_PREC)
            u = (jnp.dot(h, Wu1h, preferred_element_type=jnp.float32, precision=_PREC)
                 + jnp.dot(agg, Wu1a, preferred_element_type=jnp.float32, precision=_PREC) + bu1)
            h = jnp.dot(jnp.maximum(u, 0.0), Wu2,
                        preferred_element_type=jnp.float32, precision=_PREC) + bu2

        qp = h[0:_M, :] * qmask
        cc = h[_QS:_PAIR, :]
        tq = jnp.dot(jnp.maximum(jnp.dot(qp, W1, preferred_element_type=jnp.float32, precision=_PREC) + b1, 0.0),
                     W2, preferred_element_type=jnp.float32, precision=_PREC) + b2
        tc = jnp.dot(jnp.maximum(jnp.dot(cc, W1, preferred_element_type=jnp.float32, precision=_PREC) + b1, 0.0),
                     W2, preferred_element_type=jnp.float32, precision=_PREC) + b2
        mq = tq * qmask
        sink = lax.dot_general(mq, tc, (((1,), (1,)), ((), ())),
                               preferred_element_type=jnp.float32, precision=_PREC)
        la = sink / _TEMP
        for _s in range(_SINK_ITERS):
            mx = jnp.max(la, axis=1, keepdims=True)
            la = la - (mx + jnp.log(jnp.sum(jnp.exp(la - mx), axis=1, keepdims=True)))
            mx = jnp.max(la, axis=0, keepdims=True)
            la = la - (mx + jnp.log(jnp.sum(jnp.exp(la - mx), axis=0, keepdims=True)))
        plan = jnp.exp(la)
        temp_col = jnp.sum(plan, axis=1, keepdims=True) * qmask
        mask_col = jnp.concatenate(
            [jnp.ones((_QS, 1), jnp.float32), temp_col], axis=0)

    pc = jnp.dot(plan, cc, preferred_element_type=jnp.float32, precision=_PREC)
    score = -jnp.sum(jnp.maximum(qp - pc, 0.0))
    out_ref[...] = jnp.full((1, 1, 128), score, jnp.float32)


def kernel(node_features, edge_features, Wn, bn, We, be, Wm1, bm1, Wm2, bm2,
           Wu1, bu1, Wu2, bu2, W1, b1, W2, b2, from_idx, to_idx, graph_idx):
    del graph_idx
    fic = from_idx.reshape(_B, _EP, 1)
    tic = to_idx.reshape(_B, _EP, 1)
    tir = to_idx.reshape(_B, 1, _EP)

    def row(v):
        return v.reshape(1, -1)

    grid_spec = pl.GridSpec(
        grid=(_B,),
        in_specs=[
            pl.BlockSpec((_PAIR, 32), lambda b: (b, 0)),
            pl.BlockSpec((_EP, 16), lambda b: (b, 0)),
            pl.BlockSpec((1, _EP, 1), lambda b: (b, 0, 0)),
            pl.BlockSpec((1, _EP, 1), lambda b: (b, 0, 0)),
            pl.BlockSpec((1, 1, _EP), lambda b: (b, 0, 0)),
            pl.BlockSpec((32, 64), lambda b: (0, 0)),
            pl.BlockSpec((1, 64), lambda b: (0, 0)),
            pl.BlockSpec((16, 16), lambda b: (0, 0)),
            pl.BlockSpec((1, 16), lambda b: (0, 0)),
            pl.BlockSpec((144, 64), lambda b: (0, 0)),
            pl.BlockSpec((1, 64), lambda b: (0, 0)),
            pl.BlockSpec((64, 64), lambda b: (0, 0)),
            pl.BlockSpec((1, 64), lambda b: (0, 0)),
            pl.BlockSpec((128, 64), lambda b: (0, 0)),
            pl.BlockSpec((1, 64), lambda b: (0, 0)),
            pl.BlockSpec((64, 64), lambda b: (0, 0)),
            pl.BlockSpec((1, 64), lambda b: (0, 0)),
            pl.BlockSpec((64, 64), lambda b: (0, 0)),
            pl.BlockSpec((1, 64), lambda b: (0, 0)),
            pl.BlockSpec((64, 64), lambda b: (0, 0)),
            pl.BlockSpec((1, 64), lambda b: (0, 0)),
        ],
        out_specs=pl.BlockSpec((1, 1, 128), lambda b: (b, 0, 0)),
    )
    out = pl.pallas_call(
        _pair_body,
        grid_spec=grid_spec,
        out_shape=jax.ShapeDtypeStruct((_B, 1, 128), jnp.float32),
    )(node_features, edge_features, fic, tic, tir,
      Wn, row(bn), We, row(be), Wm1, row(bm1), Wm2, row(bm2),
      Wu1, row(bu1), Wu2, row(bu2), W1, row(b1), W2, row(b2))
    return out[:, 0, 0]


# speedup vs baseline: 8.3813x; 3.2586x over previous
import jax
import jax.numpy as jnp
from jax import lax
from jax.experimental import pallas as pl

_B = 128
_QS = 48
_CS = 64
_M = 64
_PAIR = _QS + _CS
_DEG = 16
_EP = _PAIR * _DEG
_N_PROP = 3
_N_OUTER = 5
_SINK_ITERS = 10
_TEMP = 0.1
_P = 4
_G = _B // _P
_NP = _P * _PAIR
_EPP = _P * _EP
_HIGH = lax.Precision.HIGHEST
_TN = (((0,), (0,)), ((), ()))
_NT = (((1,), (1,)), ((), ()))


def _hilo(x):
    hi = x.astype(jnp.bfloat16)
    r = x - hi.astype(jnp.float32)
    lo = r.astype(jnp.bfloat16)
    lolo = (r - lo.astype(jnp.float32)).astype(jnp.bfloat16)
    return hi, lo, lolo


def _dot2(onehot_b, x):
    hi, lo, lolo = _hilo(x)
    return (jnp.dot(onehot_b, hi, preferred_element_type=jnp.float32)
            + jnp.dot(onehot_b, lo, preferred_element_type=jnp.float32)
            + jnp.dot(onehot_b, lolo, preferred_element_type=jnp.float32))


def _dot2_tn(onehot_b, x):
    hi, lo, lolo = _hilo(x)
    return (lax.dot_general(onehot_b, hi, _TN, preferred_element_type=jnp.float32)
            + lax.dot_general(onehot_b, lo, _TN, preferred_element_type=jnp.float32)
            + lax.dot_general(onehot_b, lolo, _TN, preferred_element_type=jnp.float32))


def _dotw(x, w, prec=_HIGH):
    return jnp.dot(x, w, preferred_element_type=jnp.float32, precision=prec)


def _pair_body(nf_ref, ef_ref, fic_ref, tic_ref,
               Wn_ref, bn_ref, We_ref, be_ref, Wm1_ref, bm1_ref, Wm2_ref,
               bm2_ref, Wu1_ref, bu1_ref, Wu2_ref, bu2_ref, W1_ref, b1_ref,
               W2_ref, b2_ref, out_ref):
    step = pl.program_id(0)

    Wm1 = Wm1_ref[...]
    Wm1ab = jnp.concatenate([Wm1[0:64, :], Wm1[64:128, :]], axis=1)
    Wm1e = Wm1[128:144, :]
    Wm2 = Wm2_ref[...]
    bm2 = bm2_ref[...]
    Wu1 = Wu1_ref[...]
    Wu1h = Wu1[0:64, :]
    Wu1a = Wu1[64:128, :]
    bu1 = bu1_ref[...]
    Wu2 = Wu2_ref[...]
    bu2 = bu2_ref[...]
    W1 = W1_ref[...]
    b1 = b1_ref[...]
    W2 = W2_ref[...]
    b2 = b2_ref[...]

    h0 = _dotw(nf_ref[...], Wn_ref[...]) + bn_ref[...]
    e_lin = _dotw(ef_ref[...], We_ref[...]) + be_ref[...]
    e_term = _dotw(e_lin, Wm1e) + bm1_ref[...]

    fic_all = fic_ref[0]
    tic_all = tic_ref[0]
    iota224 = lax.broadcasted_iota(jnp.int32, (_EP, 2 * _PAIR), 1)
    iota112 = lax.broadcasted_iota(jnp.int32, (_EP, _PAIR), 1)
    Gb = []
    Tb = []
    for p in range(_P):
        base = (step * _P + p) * _PAIR
        fic = fic_all[p * _EP:(p + 1) * _EP] - base
        tic = tic_all[p * _EP:(p + 1) * _EP] - base
        g = jnp.logical_or(fic == iota224, (tic + _PAIR) == iota224)
        Gb.append(g.astype(jnp.bfloat16))
        Tb.append((tic == iota112).astype(jnp.bfloat16))

    h0AB = _dotw(h0, Wm1ab)
    R0 = []
    for p in range(_P):
        hab = h0AB[p * _PAIR:(p + 1) * _PAIR, :]
        hcat = jnp.concatenate([hab[:, 0:64], hab[:, 64:128]], axis=0)
        R0.append(jnp.maximum(_dot2(Gb[p], hcat)
                              + e_term[p * _EP:(p + 1) * _EP, :], 0.0))

    qmask = (lax.broadcasted_iota(jnp.int32, (_M, 1), 0) < _QS).astype(jnp.float32)
    qmask3 = (lax.broadcasted_iota(jnp.int32, (_P, _M, 1), 1) < _QS).astype(jnp.float32)
    ones_mf = jnp.ones((_EP, 1), jnp.float32)
    I112 = (lax.broadcasted_iota(jnp.int32, (_PAIR, _PAIR), 0)
            == lax.broadcasted_iota(jnp.int32, (_PAIR, _PAIR), 1)).astype(jnp.float32)

    plan3 = None
    qp_list = cc_list = None
    mf = [ones_mf] * _P
    for j in range(_N_OUTER):
        h = h0
        for i in range(_N_PROP):
            if i == 0:
                R = R0
            else:
                hAB = _dotw(h, Wm1ab)
                R = []
                for p in range(_P):
                    hab = hAB[p * _PAIR:(p + 1) * _PAIR, :]
                    hcat = jnp.concatenate([hab[:, 0:64], hab[:, 64:128]], axis=0)
                    R.append(jnp.maximum(_dot2(Gb[p], hcat)
                                         + e_term[p * _EP:(p + 1) * _EP, :], 0.0))
            agg_parts = []
            s_parts = []
            for p in range(_P):
                rm = R[p] if j == 0 else R[p] * mf[p]
                rma = jnp.concatenate([rm, mf[p]], axis=1)
                a = _dot2_tn(Tb[p], rma)
                agg_parts.append(a[:, 0:64])
                s_parts.append(a[:, 64:65])
            agg1 = jnp.concatenate(agg_parts, axis=0)
            s = jnp.concatenate(s_parts, axis=0)
            agg = _dotw(agg1, Wm2) + s * bm2
            u = _dotw(h, Wu1h) + _dotw(agg, Wu1a) + bu1
            h = _dotw(jnp.maximum(u, 0.0), Wu2) + bu2

        qp_list = []
        cc_list = []
        for p in range(_P):
            hp = h[p * _PAIR:(p + 1) * _PAIR, :]
            qp_list.append(hp[0:_M, :] * qmask)
            cc_list.append(hp[_QS:_PAIR, :])
        qc = jnp.concatenate(qp_list + cc_list, axis=0)
        t_qc = _dotw(jnp.maximum(_dotw(qc, W1) + b1, 0.0), W2) + b2
        sink_parts = []
        for p in range(_P):
            mq = t_qc[p * _M:(p + 1) * _M, :] * qmask
            tc = t_qc[(_P + p) * _M:(_P + p + 1) * _M, :]
            sink_parts.append(lax.dot_general(
                mq, tc, _NT, preferred_element_type=jnp.float32,
                precision=_HIGH).reshape(1, _M, _M))
        la = jnp.concatenate(sink_parts, axis=0) / _TEMP
        for _s in range(_SINK_ITERS):
            mx = jnp.max(la, axis=2, keepdims=True)
            la = la - (mx + jnp.log(jnp.sum(jnp.exp(la - mx), axis=2, keepdims=True)))
            mx = jnp.max(la, axis=1, keepdims=True)
            la = la - (mx + jnp.log(jnp.sum(jnp.exp(la - mx), axis=1, keepdims=True)))
        plan3 = jnp.exp(la)
        temp3 = jnp.sum(plan3, axis=2, keepdims=True) * qmask3
        if j < _N_OUTER - 1:
            mf = []
            for p in range(_P):
                mask_col = jnp.concatenate(
                    [jnp.ones((_QS, 1), jnp.float32), temp3[p]], axis=0)
                m_row = lax.dot_general(mask_col, I112, _TN,
                                        preferred_element_type=jnp.float32,
                                        precision=_HIGH)
                f32 = Gb[p][:, 0:_PAIR].astype(jnp.float32)
                mf.append(jnp.sum(f32 * m_row, axis=1, keepdims=True))

    score_vec = jnp.zeros((1, 1, 128), jnp.float32)
    lane = lax.broadcasted_iota(jnp.int32, (1, 1, 128), 2)
    for p in range(_P):
        pc = _dotw(plan3[p], cc_list[p])
        sc = -jnp.sum(jnp.maximum(qp_list[p] - pc, 0.0))
        score_vec = jnp.where(lane == p, sc, score_vec)
    out_ref[...] = score_vec


def kernel(node_features, edge_features, Wn, bn, We, be, Wm1, bm1, Wm2, bm2,
           Wu1, bu1, Wu2, bu2, W1, b1, W2, b2, from_idx, to_idx, graph_idx):
    del graph_idx
    fic = from_idx.reshape(_G, _EPP, 1)
    tic = to_idx.reshape(_G, _EPP, 1)

    def row(v):
        return v.reshape(1, -1)

    grid_spec = pl.GridSpec(
        grid=(_G,),
        in_specs=[
            pl.BlockSpec((_NP, 32), lambda b: (b, 0)),
            pl.BlockSpec((_EPP, 16), lambda b: (b, 0)),
            pl.BlockSpec((1, _EPP, 1), lambda b: (b, 0, 0)),
            pl.BlockSpec((1, _EPP, 1), lambda b: (b, 0, 0)),
            pl.BlockSpec((32, 64), lambda b: (0, 0)),
            pl.BlockSpec((1, 64), lambda b: (0, 0)),
            pl.BlockSpec((16, 16), lambda b: (0, 0)),
            pl.BlockSpec((1, 16), lambda b: (0, 0)),
            pl.BlockSpec((144, 64), lambda b: (0, 0)),
            pl.BlockSpec((1, 64), lambda b: (0, 0)),
            pl.BlockSpec((64, 64), lambda b: (0, 0)),
            pl.BlockSpec((1, 64), lambda b: (0, 0)),
            pl.BlockSpec((128, 64), lambda b: (0, 0)),
            pl.BlockSpec((1, 64), lambda b: (0, 0)),
            pl.BlockSpec((64, 64), lambda b: (0, 0)),
            pl.BlockSpec((1, 64), lambda b: (0, 0)),
            pl.BlockSpec((64, 64), lambda b: (0, 0)),
            pl.BlockSpec((1, 64), lambda b: (0, 0)),
            pl.BlockSpec((64, 64), lambda b: (0, 0)),
            pl.BlockSpec((1, 64), lambda b: (0, 0)),
        ],
        out_specs=pl.BlockSpec((1, 1, 128), lambda b: (b, 0, 0)),
    )
    out = pl.pallas_call(
        _pair_body,
        grid_spec=grid_spec,
        out_shape=jax.ShapeDtypeStruct((_G, 1, 128), jnp.float32),
    )(node_features, edge_features, fic, tic,
      Wn, row(bn), We, row(be), Wm1, row(bm1), Wm2, row(bm2),
      Wu1, row(bu1), Wu2, row(bu2), W1, row(b1), W2, row(b2))
    return out[:, 0, 0:_P].reshape(_B)
